# baseline (speedup 1.0000x reference)
import jax
import jax.numpy as jnp
from jax.experimental import pallas as pl
from jax.experimental.pallas import tpu as pltpu

B = 2
Q = 900
C = 2
K = 100
NKP = 144
NVERT = 10475
NPOSE = 159


def _select_body(logits_ref, boxes_ref, lh_ref, rh_ref, fc_ref, pose_ref,
                 beta_ref, expr_ref, cam_ref, kp3d_ref, ts_ref, img_ref,
                 scores_ref, labels_ref, tk_ref, boxes_o_ref, lh_o_ref,
                 rh_o_ref, fc_o_ref, pose_o_ref, beta_o_ref, expr_o_ref,
                 transl_o_ref, kp3d_o_ref, kp2d_o_ref):
    flat = (jax.lax.broadcasted_iota(jnp.int32, (16, 128), 1) * 16
            + jax.lax.broadcasted_iota(jnp.int32, (16, 128), 0))
    lane128 = jax.lax.broadcasted_iota(jnp.int32, (1, 128), 1)
    sub16 = jax.lax.broadcasted_iota(jnp.int32, (16, 128), 0)
    q_iota = jax.lax.broadcasted_iota(jnp.int32, (Q, 128), 0)

    def colsort(p, fl):
        for kk in (2, 4, 8, 16):
            jj = kk // 2
            while jj >= 1:
                pp = pltpu.roll(p, jj, axis=0)
                pm = pltpu.roll(p, 16 - jj, axis=0)
                fp = pltpu.roll(fl, jj, axis=0)
                fm = pltpu.roll(fl, 16 - jj, axis=0)
                up = (sub16 & jj) == 0
                pv = jnp.where(up, pm, pp)
                pf = jnp.where(up, fm, fp)
                tm = ((sub16 & kk) == 0) == up
                sf = (p > pv) | ((p == pv) & (fl < pf))
                keep = (sf & tm) | (~sf & ~tm)
                p = jnp.where(keep, p, pv)
                fl = jnp.where(keep, fl, pf)
                jj //= 2
        return p, fl

    def step(p, fl, s_row, i_row, k):
        head = p[0:1, :]
        hf = fl[0:1, :]
        m = jnp.max(head, axis=1, keepdims=True)
        idx = jnp.min(jnp.where(head == m, hf, 99999),
                      axis=1, keepdims=True)
        s_row = jnp.where(lane128 == k, m, s_row)
        i_row = jnp.where(lane128 == k, idx, i_row)
        col = hf == idx
        adv_p = jnp.where(sub16 == 15, -2.0, pltpu.roll(p, 15, axis=0))
        adv_f = pltpu.roll(fl, 15, axis=0)
        p = jnp.where(col, adv_p, p)
        fl = jnp.where(col, adv_f, fl)
        return p, fl, s_row, i_row

    def body(k, carry):
        p0, f0, s0, i0, p1, f1, s1, i1 = carry
        p0, f0, s0, i0 = step(p0, f0, s0, i0, k)
        p1, f1, s1, i1 = step(p1, f1, s1, i1, k)
        return p0, f0, s0, i0, p1, f1, s1, i1

    def prep(x):
        return jnp.where(flat < Q * C, jax.nn.sigmoid(x), -2.0)

    ps0, fl0 = colsort(prep(logits_ref[0]), flat)
    ps1, fl1 = colsort(prep(logits_ref[1]), flat)
    init = (ps0, fl0,
            jnp.zeros((1, 128), jnp.float32),
            jnp.full((1, 128), -1, jnp.int32),
            ps1, fl1,
            jnp.zeros((1, 128), jnp.float32),
            jnp.full((1, 128), -1, jnp.int32))
    _, _, s0, i0, _, _, s1, i1 = jax.lax.fori_loop(0, K, body, init)
    topk_rows = ((s0, i0), (s1, i1))

    for b in range(B):
        s_row, i_row = topk_rows[b]

        scores_ref[b, :] = s_row[0, :K]
        labels_ref[b, :] = jnp.where(i_row[0, :K] >= 0, i_row[0, :K] % C, 0)
        tkq = jnp.where(i_row >= 0, i_row // C, 0)
        tk_ref[b, :] = tkq[0, :K]

        onehot = jnp.where(q_iota == tkq, 1.0, 0.0).astype(jnp.float32)

        def mm(x):
            return jax.lax.dot_general(
                x, onehot, (((1,), (0,)), ((), ())),
                precision=jax.lax.Precision.HIGHEST,
                preferred_element_type=jnp.float32)

        img_h = ts_ref[b, 0]
        img_w = ts_ref[b, 1]

        for ref, oref in ((boxes_ref, boxes_o_ref), (lh_ref, lh_o_ref),
                          (rh_ref, rh_o_ref), (fc_ref, fc_o_ref)):
            g = mm(ref[b])
            cx, cy, w, h = g[0:1], g[1:2], g[2:3], g[3:4]
            rows = jnp.concatenate([
                (cx - 0.5 * w) * img_w,
                (cy - 0.5 * h) * img_h,
                (cx + 0.5 * w) * img_w,
                (cy + 0.5 * h) * img_h,
            ], axis=0)
            oref[b, :, :] = rows[:, :K]

        pose_o_ref[:, b, :] = mm(pose_ref[:, b, :])[:, :K]
        beta_o_ref[:, b, :] = mm(beta_ref[:, b, :])[:, :K]
        expr_o_ref[:, b, :] = mm(expr_ref[:, b, :])[:, :K]

        gcam = mm(cam_ref[:, b, :])
        s = gcam[0:1] + 1e-9
        txs = gcam[1:2] / s
        tys = gcam[2:3] / s
        invs = 1.0 / s
        transl_o_ref[:, b, :] = jnp.concatenate([txs, tys, invs],
                                                axis=0)[:, :K]

        k3 = kp3d_ref[b]
        gx = mm(k3[0])
        gy = mm(k3[1])
        gz = mm(k3[2])
        kp3d_o_ref[b, 0, :, :] = gx[:, :K]
        kp3d_o_ref[b, 1, :, :] = gy[:, :K]
        kp3d_o_ref[b, 2, :, :] = gz[:, :K]

        cc_x = img_ref[b, 1] * 0.5
        cc_y = img_ref[b, 0] * 0.5
        zz = gz + invs
        kp2d_o_ref[b, 0, :, :] = ((gx + txs) / zz * 5000.0 + cc_x)[:, :K]
        kp2d_o_ref[b, 1, :, :] = ((gy + tys) / zz * 5000.0 + cc_y)[:, :K]


GS = 20


def _gather_body(idx_ref, *refs):
    a_refs = refs[0:GS]
    b_refs = refs[GS:2 * GS]
    out_ref = refs[2 * GS]
    for j in range(GS):
        out_ref[j, :, 0, :] = a_refs[j][0, :, 0, :]
        out_ref[j, :, 1, :] = b_refs[j][0, :, 1, :]


def kernel(pred_logits, pred_boxes, pred_lhand_boxes, pred_rhand_boxes,
           pred_face_boxes, pred_smpl_fullpose, pred_smpl_beta,
           pred_smpl_expr, pred_smpl_cam, pred_smpl_kp3d, pred_smpl_verts,
           target_sizes, img_shape):
    lf = jnp.pad(pred_logits.reshape(B, Q * C), ((0, 0), (0, 2048 - Q * C)))
    l16 = jnp.transpose(lf.reshape(B, 128, 16), (0, 2, 1))
    boxes_t = jnp.transpose(pred_boxes, (0, 2, 1))
    lh_t = jnp.transpose(pred_lhand_boxes, (0, 2, 1))
    rh_t = jnp.transpose(pred_rhand_boxes, (0, 2, 1))
    fc_t = jnp.transpose(pred_face_boxes, (0, 2, 1))
    pose_t = jnp.transpose(pred_smpl_fullpose, (2, 0, 1))
    beta_t = jnp.transpose(pred_smpl_beta, (2, 0, 1))
    expr_t = jnp.transpose(pred_smpl_expr, (2, 0, 1))
    cam_t = jnp.transpose(pred_smpl_cam, (2, 0, 1))
    kp3d_t = jnp.transpose(pred_smpl_kp3d, (0, 3, 2, 1))
    verts_t = jnp.transpose(pred_smpl_verts, (1, 3, 0, 2))

    full = lambda shape: pl.BlockSpec(shape, lambda: tuple(0 for _ in shape))
    in_specs = [
        full((B, 16, 128)),
        full((B, 4, Q)),
        full((B, 4, Q)),
        full((B, 4, Q)),
        full((B, 4, Q)),
        full((NPOSE, B, Q)),
        full((10, B, Q)),
        full((10, B, Q)),
        full((3, B, Q)),
        full((B, 3, NKP, Q)),
        full((B, 2)),
        full((B, 2)),
    ]
    out_shape = (
        jax.ShapeDtypeStruct((B, K), jnp.float32),
        jax.ShapeDtypeStruct((B, K), jnp.int32),
        jax.ShapeDtypeStruct((B, K), jnp.int32),
        jax.ShapeDtypeStruct((B, 4, K), jnp.float32),
        jax.ShapeDtypeStruct((B, 4, K), jnp.float32),
        jax.ShapeDtypeStruct((B, 4, K), jnp.float32),
        jax.ShapeDtypeStruct((B, 4, K), jnp.float32),
        jax.ShapeDtypeStruct((NPOSE, B, K), jnp.float32),
        jax.ShapeDtypeStruct((10, B, K), jnp.float32),
        jax.ShapeDtypeStruct((10, B, K), jnp.float32),
        jax.ShapeDtypeStruct((3, B, K), jnp.float32),
        jax.ShapeDtypeStruct((B, 3, NKP, K), jnp.float32),
        jax.ShapeDtypeStruct((B, 2, NKP, K), jnp.float32),
    )
    out_specs = [full(s.shape) for s in out_shape]
    (scores, labels, tk, boxes_o, lh_o, rh_o, fc_o, pose_o, beta_o, expr_o,
     transl_o, kp3d_o, kp2d_o) = pl.pallas_call(
        _select_body,
        in_specs=in_specs,
        out_specs=out_specs,
        out_shape=out_shape,
    )(l16, boxes_t, lh_t, rh_t, fc_t, pose_t, beta_t, expr_t, cam_t,
      kp3d_t, target_sizes, img_shape)

    def in_map(b, j):
        return lambda i, idx: (idx[b, GS * i + j], 0, 0, 0)

    vspec = (1, 3, B, NVERT)
    grid_spec = pltpu.PrefetchScalarGridSpec(
        num_scalar_prefetch=1,
        grid=(K // GS,),
        in_specs=[pl.BlockSpec(vspec, in_map(0, j)) for j in range(GS)]
                 + [pl.BlockSpec(vspec, in_map(1, j)) for j in range(GS)],
        out_specs=pl.BlockSpec((GS, 3, B, NVERT),
                               lambda i, idx: (i, 0, 0, 0)),
    )
    verts_sel_t = pl.pallas_call(
        _gather_body,
        grid_spec=grid_spec,
        out_shape=jax.ShapeDtypeStruct((K, 3, B, NVERT), jnp.float32),
    )(tk, *([verts_t] * (2 * GS)))

    smpl_verts = jnp.transpose(verts_sel_t, (2, 0, 3, 1))
    kp3d = jnp.transpose(kp3d_o, (0, 3, 2, 1))
    kp2d = jnp.transpose(kp2d_o, (0, 3, 2, 1))
    poseg = jnp.transpose(pose_o, (1, 2, 0))
    betag = jnp.transpose(beta_o, (1, 2, 0))
    exprg = jnp.transpose(expr_o, (1, 2, 0))
    transl = jnp.transpose(transl_o, (1, 2, 0))
    bsel = jnp.transpose(boxes_o, (0, 2, 1))
    lho = jnp.transpose(lh_o, (0, 2, 1))
    rho = jnp.transpose(rh_o, (0, 2, 1))
    fco = jnp.transpose(fc_o, (0, 2, 1))

    root_pose = poseg[:, :, :3]
    body_pose = poseg[:, :, 3:66]
    lhand_pose = poseg[:, :, 66:111]
    rhand_pose = poseg[:, :, 111:156]
    jaw_pose = poseg[:, :, 156:]

    return (scores, labels, kp3d, root_pose, body_pose, lhand_pose,
            rhand_pose, jaw_pose, betag, exprg, kp2d, smpl_verts, transl,
            bsel, lho, rho, fco, bsel)


# speedup vs baseline: 1.3445x; 1.3445x over previous
import jax
import jax.numpy as jnp
from jax.experimental import pallas as pl
from jax.experimental.pallas import tpu as pltpu

B = 2
Q = 900
C = 2
K = 100
NKP = 144
NVERT = 10475
NPOSE = 159


def _select_body(logits_ref, boxes_ref, lh_ref, rh_ref, fc_ref, pose_ref,
                 beta_ref, expr_ref, cam_ref, kp3d_ref, ts_ref, img_ref,
                 scores_ref, labels_ref, tk_ref, boxes_o_ref, lh_o_ref,
                 rh_o_ref, fc_o_ref, pose_o_ref, beta_o_ref, expr_o_ref,
                 transl_o_ref, kp3d_o_ref, kp2d_o_ref):
    flat = (jax.lax.broadcasted_iota(jnp.int32, (16, 128), 0) * 128
            + jax.lax.broadcasted_iota(jnp.int32, (16, 128), 1))
    lane128 = jax.lax.broadcasted_iota(jnp.int32, (1, 128), 1)
    q_iota = jax.lax.broadcasted_iota(jnp.int32, (Q, 128), 0)

    def partner(x, j):
        if j >= 128:
            r = j // 128
            plus = pltpu.roll(x, 16 - r, axis=0)
            minus = pltpu.roll(x, r, axis=0)
        else:
            plus = pltpu.roll(x, 128 - j, axis=1)
            minus = pltpu.roll(x, j, axis=1)
        return plus, minus

    def bitonic(p, fl):
        kk = 2
        while kk <= 2048:
            jj = kk // 2
            while jj >= 1:
                pvp, pvm = partner(p, jj)
                pfp, pfm = partner(fl, jj)
                up = (flat & jj) == 0
                pv = jnp.where(up, pvp, pvm)
                pf = jnp.where(up, pfp, pfm)
                tm = ((flat & kk) == 0) == up
                sf = (p > pv) | ((p == pv) & (fl < pf))
                keep = (sf & tm) | (~sf & ~tm)
                p = jnp.where(keep, p, pv)
                fl = jnp.where(keep, fl, pf)
                jj //= 2
            kk *= 2
        return p, fl

    def prep(x):
        return jnp.where(flat < Q * C, jax.nn.sigmoid(x), -2.0)

    ps0, fl0 = bitonic(prep(logits_ref[0]), flat)
    ps1, fl1 = bitonic(prep(logits_ref[1]), flat)
    topk_rows = ((ps0[0:1, :], fl0[0:1, :]), (ps1[0:1, :], fl1[0:1, :]))

    for b in range(B):
        s_row, i_row = topk_rows[b]

        scores_ref[b, :] = s_row[0, :K]
        labels_ref[b, :] = jnp.where(i_row[0, :K] >= 0, i_row[0, :K] % C, 0)
        tkq = jnp.where(i_row >= 0, i_row // C, 0)
        tk_ref[b, :] = tkq[0, :K]

        onehot = jnp.where(q_iota == tkq, 1.0, 0.0).astype(jnp.float32)

        def mm(x):
            return jax.lax.dot_general(
                x, onehot, (((1,), (0,)), ((), ())),
                precision=jax.lax.Precision.HIGHEST,
                preferred_element_type=jnp.float32)

        img_h = ts_ref[b, 0]
        img_w = ts_ref[b, 1]

        for ref, oref in ((boxes_ref, boxes_o_ref), (lh_ref, lh_o_ref),
                          (rh_ref, rh_o_ref), (fc_ref, fc_o_ref)):
            g = mm(ref[b])
            cx, cy, w, h = g[0:1], g[1:2], g[2:3], g[3:4]
            rows = jnp.concatenate([
                (cx - 0.5 * w) * img_w,
                (cy - 0.5 * h) * img_h,
                (cx + 0.5 * w) * img_w,
                (cy + 0.5 * h) * img_h,
            ], axis=0)
            oref[b, :, :] = rows[:, :K]

        pose_o_ref[:, b, :] = mm(pose_ref[:, b, :])[:, :K]
        beta_o_ref[:, b, :] = mm(beta_ref[:, b, :])[:, :K]
        expr_o_ref[:, b, :] = mm(expr_ref[:, b, :])[:, :K]

        gcam = mm(cam_ref[:, b, :])
        s = gcam[0:1] + 1e-9
        txs = gcam[1:2] / s
        tys = gcam[2:3] / s
        invs = 1.0 / s
        transl_o_ref[:, b, :] = jnp.concatenate([txs, tys, invs],
                                                axis=0)[:, :K]

        k3 = kp3d_ref[b]
        gx = mm(k3[0])
        gy = mm(k3[1])
        gz = mm(k3[2])
        kp3d_o_ref[b, 0, :, :] = gx[:, :K]
        kp3d_o_ref[b, 1, :, :] = gy[:, :K]
        kp3d_o_ref[b, 2, :, :] = gz[:, :K]

        cc_x = img_ref[b, 1] * 0.5
        cc_y = img_ref[b, 0] * 0.5
        zz = gz + invs
        kp2d_o_ref[b, 0, :, :] = ((gx + txs) / zz * 5000.0 + cc_x)[:, :K]
        kp2d_o_ref[b, 1, :, :] = ((gy + tys) / zz * 5000.0 + cc_y)[:, :K]


GS = 20


def _gather_body(idx_ref, *refs):
    a_refs = refs[0:GS]
    b_refs = refs[GS:2 * GS]
    out_ref = refs[2 * GS]
    for j in range(GS):
        out_ref[j, :, 0, :] = a_refs[j][0, :, 0, :]
        out_ref[j, :, 1, :] = b_refs[j][0, :, 1, :]


def kernel(pred_logits, pred_boxes, pred_lhand_boxes, pred_rhand_boxes,
           pred_face_boxes, pred_smpl_fullpose, pred_smpl_beta,
           pred_smpl_expr, pred_smpl_cam, pred_smpl_kp3d, pred_smpl_verts,
           target_sizes, img_shape):
    lf = jnp.pad(pred_logits.reshape(B, Q * C), ((0, 0), (0, 2048 - Q * C)))
    l16 = lf.reshape(B, 16, 128)
    boxes_t = jnp.transpose(pred_boxes, (0, 2, 1))
    lh_t = jnp.transpose(pred_lhand_boxes, (0, 2, 1))
    rh_t = jnp.transpose(pred_rhand_boxes, (0, 2, 1))
    fc_t = jnp.transpose(pred_face_boxes, (0, 2, 1))
    pose_t = jnp.transpose(pred_smpl_fullpose, (2, 0, 1))
    beta_t = jnp.transpose(pred_smpl_beta, (2, 0, 1))
    expr_t = jnp.transpose(pred_smpl_expr, (2, 0, 1))
    cam_t = jnp.transpose(pred_smpl_cam, (2, 0, 1))
    kp3d_t = jnp.transpose(pred_smpl_kp3d, (0, 3, 2, 1))
    verts_t = jnp.transpose(pred_smpl_verts, (1, 3, 0, 2))

    full = lambda shape: pl.BlockSpec(shape, lambda: tuple(0 for _ in shape))
    in_specs = [
        full((B, 16, 128)),
        full((B, 4, Q)),
        full((B, 4, Q)),
        full((B, 4, Q)),
        full((B, 4, Q)),
        full((NPOSE, B, Q)),
        full((10, B, Q)),
        full((10, B, Q)),
        full((3, B, Q)),
        full((B, 3, NKP, Q)),
        full((B, 2)),
        full((B, 2)),
    ]
    out_shape = (
        jax.ShapeDtypeStruct((B, K), jnp.float32),
        jax.ShapeDtypeStruct((B, K), jnp.int32),
        jax.ShapeDtypeStruct((B, K), jnp.int32),
        jax.ShapeDtypeStruct((B, 4, K), jnp.float32),
        jax.ShapeDtypeStruct((B, 4, K), jnp.float32),
        jax.ShapeDtypeStruct((B, 4, K), jnp.float32),
        jax.ShapeDtypeStruct((B, 4, K), jnp.float32),
        jax.ShapeDtypeStruct((NPOSE, B, K), jnp.float32),
        jax.ShapeDtypeStruct((10, B, K), jnp.float32),
        jax.ShapeDtypeStruct((10, B, K), jnp.float32),
        jax.ShapeDtypeStruct((3, B, K), jnp.float32),
        jax.ShapeDtypeStruct((B, 3, NKP, K), jnp.float32),
        jax.ShapeDtypeStruct((B, 2, NKP, K), jnp.float32),
    )
    out_specs = [full(s.shape) for s in out_shape]
    (scores, labels, tk, boxes_o, lh_o, rh_o, fc_o, pose_o, beta_o, expr_o,
     transl_o, kp3d_o, kp2d_o) = pl.pallas_call(
        _select_body,
        in_specs=in_specs,
        out_specs=out_specs,
        out_shape=out_shape,
    )(l16, boxes_t, lh_t, rh_t, fc_t, pose_t, beta_t, expr_t, cam_t,
      kp3d_t, target_sizes, img_shape)

    def in_map(b, j):
        return lambda i, idx: (idx[b, GS * i + j], 0, 0, 0)

    vspec = (1, 3, B, NVERT)
    grid_spec = pltpu.PrefetchScalarGridSpec(
        num_scalar_prefetch=1,
        grid=(K // GS,),
        in_specs=[pl.BlockSpec(vspec, in_map(0, j)) for j in range(GS)]
                 + [pl.BlockSpec(vspec, in_map(1, j)) for j in range(GS)],
        out_specs=pl.BlockSpec((GS, 3, B, NVERT),
                               lambda i, idx: (i, 0, 0, 0)),
    )
    verts_sel_t = pl.pallas_call(
        _gather_body,
        grid_spec=grid_spec,
        out_shape=jax.ShapeDtypeStruct((K, 3, B, NVERT), jnp.float32),
    )(tk, *([verts_t] * (2 * GS)))

    smpl_verts = jnp.transpose(verts_sel_t, (2, 0, 3, 1))
    kp3d = jnp.transpose(kp3d_o, (0, 3, 2, 1))
    kp2d = jnp.transpose(kp2d_o, (0, 3, 2, 1))
    poseg = jnp.transpose(pose_o, (1, 2, 0))
    betag = jnp.transpose(beta_o, (1, 2, 0))
    exprg = jnp.transpose(expr_o, (1, 2, 0))
    transl = jnp.transpose(transl_o, (1, 2, 0))
    bsel = jnp.transpose(boxes_o, (0, 2, 1))
    lho = jnp.transpose(lh_o, (0, 2, 1))
    rho = jnp.transpose(rh_o, (0, 2, 1))
    fco = jnp.transpose(fc_o, (0, 2, 1))

    root_pose = poseg[:, :, :3]
    body_pose = poseg[:, :, 3:66]
    lhand_pose = poseg[:, :, 66:111]
    rhand_pose = poseg[:, :, 111:156]
    jaw_pose = poseg[:, :, 156:]

    return (scores, labels, kp3d, root_pose, body_pose, lhand_pose,
            rhand_pose, jaw_pose, betag, exprg, kp2d, smpl_verts, transl,
            bsel, lho, rho, fco, bsel)
